# Initial kernel scaffold; baseline (speedup 1.0000x reference)
import jax
import jax.numpy as jnp
from jax.experimental import pallas as pl


def kernel(x, edge_src, edge_dst, edge_val, filterCoeff):
    raise NotImplementedError("write your pallas kernel here")


# speedup vs baseline: 53.6475x; 53.6475x over previous
import functools

import jax
import jax.numpy as jnp
from jax import lax
from jax.experimental import pallas as pl
from jax.experimental.pallas import tpu as pltpu
from jax.experimental.pallas import tpu_sc as plsc

N_NODES = 10000
D_FEAT = 128
OUT_FEAT = 8
FILTER_ORDER = 3
N_EDGES = 320000
K_CHAIN = 17

NSUB = 16
LANES = 16
CHUNK = 128
NODES_PER_TILE = 640
N_PAD = NSUB * NODES_PER_TILE
CHUNKS_PER_TILE = 160
EDGES_PER_TILE = CHUNKS_PER_TILE * CHUNK
E_PAD = NSUB * EDGES_PER_TILE
A_PAD = 144


def _rowsum_body(x_ref, o_ref):
    o_ref[...] = jnp.sum(x_ref[...], axis=1)


_rowsum = pl.pallas_call(
    _rowsum_body,
    out_shape=jax.ShapeDtypeStruct((N_NODES,), jnp.float32),
)


def _sc_chain():
    mesh = plsc.VectorSubcoreMesh(core_axis_name="c", subcore_axis_name="s")

    @functools.partial(
        pl.kernel,
        out_type=jax.ShapeDtypeStruct((N_NODES, OUT_FEAT), jnp.float32),
        mesh=mesh,
        scratch_types=[
            pltpu.VMEM((N_PAD,), jnp.float32),
            pltpu.VMEM((CHUNKS_PER_TILE, CHUNK), jnp.int32),
            pltpu.VMEM((CHUNKS_PER_TILE, CHUNK), jnp.int32),
            pltpu.VMEM((CHUNKS_PER_TILE, CHUNK), jnp.float32),
            pltpu.VMEM((CHUNKS_PER_TILE, CHUNK), jnp.float32),
            pltpu.VMEM((OUT_FEAT, NODES_PER_TILE), jnp.float32),
            pltpu.VMEM((NODES_PER_TILE, OUT_FEAT), jnp.float32),
            pltpu.VMEM((NODES_PER_TILE,), jnp.float32),
            pltpu.VMEM((A_PAD,), jnp.float32),
            pltpu.VMEM_SHARED((N_PAD,), jnp.float32),
            pltpu.SemaphoreType.DMA,
        ],
        compiler_params=pltpu.CompilerParams(needs_layout_passes=False,
                                             use_tc_tiling_on_sc=False),
    )
    def chain(s_hbm, src_hbm, dst_hbm, val_hbm, a_hbm, y_hbm,
              cur, src2, dst2, val2, prod2, yt, ytr, zeros, atab, shared, sem):
        cid = lax.axis_index("c")
        sid = lax.axis_index("s")
        ebase = sid * CHUNKS_PER_TILE
        nbase = sid * NODES_PER_TILE

        pltpu.sync_copy(s_hbm, cur)
        pltpu.sync_copy(src_hbm.at[pl.ds(ebase, CHUNKS_PER_TILE), :], src2)
        pltpu.sync_copy(dst_hbm.at[pl.ds(ebase, CHUNKS_PER_TILE), :], dst2)
        pltpu.sync_copy(val_hbm.at[pl.ds(ebase, CHUNKS_PER_TILE), :], val2)
        pltpu.sync_copy(a_hbm, atab)

        zv = jnp.zeros((LANES,), jnp.float32)

        def _zero_zeros(b, carry):
            zeros[pl.ds(b * LANES, LANES)] = zv
            return carry

        lax.fori_loop(0, NODES_PER_TILE // LANES, _zero_zeros, 0)

        for o in range(OUT_FEAT):
            def _zero_yt(b, carry, o=o):
                yt[o, pl.ds(b * LANES, LANES)] = zv
                return carry

            lax.fori_loop(0, NODES_PER_TILE // LANES, _zero_yt, 0)

        pltpu.sync_copy(zeros, shared.at[pl.ds(nbase, NODES_PER_TILE)])
        plsc.subcore_barrier()

        def chain_body(r, carry):
            def _gather(c, carry2):
                for j in range(CHUNK // LANES):
                    sl = pl.ds(j * LANES, LANES)
                    g = plsc.load_gather(cur, [src2[c, sl]])
                    prod2[c, sl] = g * val2[c, sl]
                return carry2

            lax.fori_loop(0, CHUNKS_PER_TILE, _gather, 0)

            def _fire(c, carry2):
                pltpu.async_copy(prod2.at[c], shared.at[dst2.at[c]], sem,
                                 add=True)
                return carry2

            lax.fori_loop(0, CHUNKS_PER_TILE, _fire, 0)

            def _drain(c, carry2):
                pltpu.make_async_copy(prod2.at[c], shared.at[dst2.at[c]],
                                      sem).wait()
                return carry2

            lax.fori_loop(0, CHUNKS_PER_TILE, _drain, 0)
            plsc.subcore_barrier()

            pltpu.sync_copy(shared, cur)
            for o in range(OUT_FEAT):
                av = plsc.load_gather(
                    atab, [jnp.full((LANES,), r * OUT_FEAT + o, jnp.int32)])

                def _yacc(b, carry2, o=o, av=av):
                    sl = pl.ds(b * LANES, LANES)
                    yt[o, sl] = yt[o, sl] + av * cur[pl.ds(nbase + b * LANES,
                                                           LANES)]
                    return carry2

                lax.fori_loop(0, NODES_PER_TILE // LANES, _yacc, 0)

            plsc.subcore_barrier()
            pltpu.sync_copy(zeros, shared.at[pl.ds(nbase, NODES_PER_TILE)])
            plsc.subcore_barrier()
            return carry

        lax.fori_loop(0, K_CHAIN, chain_body, 0)

        iota = lax.iota(jnp.int32, LANES)
        for o in range(OUT_FEAT):
            def _tr(b, carry2, o=o):
                v = yt[o, pl.ds(b * LANES, LANES)]
                plsc.store_scatter(
                    ytr, [iota + b * LANES, jnp.full((LANES,), o, jnp.int32)],
                    v)
                return carry2

            lax.fori_loop(0, NODES_PER_TILE // LANES, _tr, 0)

        @pl.when(cid == 0)
        def _():
            @pl.when(sid < NSUB - 1)
            def _():
                pltpu.sync_copy(ytr,
                                y_hbm.at[pl.ds(nbase, NODES_PER_TILE), :])

            @pl.when(sid == NSUB - 1)
            def _():
                last = N_NODES - (NSUB - 1) * NODES_PER_TILE
                pltpu.sync_copy(
                    ytr.at[pl.ds(0, last), :],
                    y_hbm.at[pl.ds((NSUB - 1) * NODES_PER_TILE, last), :])

    return chain


_chain_kernel = _sc_chain()


def kernel(x, edge_src, edge_dst, edge_val, filterCoeff):
    x = x.astype(jnp.float32)
    src = edge_src.astype(jnp.int32)
    dst = edge_dst.astype(jnp.int32)
    val = edge_val.astype(jnp.float32)
    coeff = filterCoeff.astype(jnp.float32)

    s = _rowsum(x)
    s_pad = jnp.pad(s, (0, N_PAD - N_NODES))

    pad = E_PAD - N_EDGES
    src2 = jnp.pad(src, (0, pad)).reshape(NSUB * CHUNKS_PER_TILE, CHUNK)
    dst2 = jnp.pad(dst, (0, pad)).reshape(NSUB * CHUNKS_PER_TILE, CHUNK)
    val2 = jnp.pad(val, (0, pad)).reshape(NSUB * CHUNKS_PER_TILE, CHUNK)

    r = jnp.arange(K_CHAIN)[:, None]
    o = jnp.arange(OUT_FEAT)[None, :]
    i = r - 2 * o
    valid = (i >= 0) & (i < FILTER_ORDER)
    atab = jnp.where(valid, coeff.T[jnp.clip(i, 0, FILTER_ORDER - 1), o], 0.0)
    atab = jnp.pad(atab.reshape(-1), (0, A_PAD - K_CHAIN * OUT_FEAT))

    return _chain_kernel(s_pad, src2, dst2, val2, atab)


# speedup vs baseline: 79.7313x; 1.4862x over previous
import functools

import jax
import jax.numpy as jnp
from jax import lax
from jax.experimental import pallas as pl
from jax.experimental.pallas import tpu as pltpu
from jax.experimental.pallas import tpu_sc as plsc

N_NODES = 10000
D_FEAT = 128
OUT_FEAT = 8
FILTER_ORDER = 3
N_EDGES = 320000
K_CHAIN = 17

NSUB = 16
LANES = 16
CHUNK = 128
NODES_PER_TILE = 640
N_PAD = NSUB * NODES_PER_TILE
CHUNKS_PER_TILE = 160
EDGES_PER_TILE = CHUNKS_PER_TILE * CHUNK
E_PAD = NSUB * EDGES_PER_TILE
A_PAD = 144
NREG = NODES_PER_TILE // LANES


def _rowsum_body(x_ref, o_ref):
    o_ref[...] = jnp.sum(x_ref[...], axis=1)


_rowsum = pl.pallas_call(
    _rowsum_body,
    out_shape=jax.ShapeDtypeStruct((N_NODES,), jnp.float32),
)


def _sc_chain():
    mesh = plsc.VectorSubcoreMesh(core_axis_name="c", subcore_axis_name="s")

    @functools.partial(
        pl.kernel,
        out_type=jax.ShapeDtypeStruct((N_NODES, OUT_FEAT), jnp.float32),
        mesh=mesh,
        scratch_types=[
            pltpu.VMEM((N_PAD,), jnp.float32),
            pltpu.VMEM((CHUNKS_PER_TILE, CHUNK), jnp.int32),
            pltpu.VMEM((CHUNKS_PER_TILE, CHUNK), jnp.int32),
            pltpu.VMEM((CHUNKS_PER_TILE, CHUNK), jnp.float32),
            pltpu.VMEM((CHUNKS_PER_TILE, CHUNK), jnp.float32),
            pltpu.VMEM((OUT_FEAT * NODES_PER_TILE,), jnp.float32),
            pltpu.VMEM((NODES_PER_TILE, OUT_FEAT), jnp.float32),
            pltpu.VMEM((NODES_PER_TILE,), jnp.float32),
            pltpu.VMEM((A_PAD,), jnp.float32),
            pltpu.VMEM_SHARED((2, N_PAD), jnp.float32),
            pltpu.SemaphoreType.DMA,
        ],
        compiler_params=pltpu.CompilerParams(needs_layout_passes=False,
                                             use_tc_tiling_on_sc=False),
    )
    def chain(s_hbm, src_hbm, dst_hbm, val_hbm, a_hbm, y_hbm,
              cur, src2, dst2, val2, prod2, yt, ytr, zeros, atab,
              shared2, sem):
        cid = lax.axis_index("c")
        sid = lax.axis_index("s")
        ebase = sid * CHUNKS_PER_TILE
        nbase = sid * NODES_PER_TILE

        pltpu.sync_copy(s_hbm, cur)
        pltpu.sync_copy(src_hbm.at[pl.ds(ebase, CHUNKS_PER_TILE), :], src2)
        pltpu.sync_copy(dst_hbm.at[pl.ds(ebase, CHUNKS_PER_TILE), :], dst2)
        pltpu.sync_copy(val_hbm.at[pl.ds(ebase, CHUNKS_PER_TILE), :], val2)
        pltpu.sync_copy(a_hbm, atab)

        zv = jnp.zeros((LANES,), jnp.float32)

        def _zero_zeros(b, carry):
            zeros[pl.ds(b * LANES, LANES)] = zv
            return carry

        lax.fori_loop(0, NREG, _zero_zeros, 0)

        def _zero_yt(b, carry):
            yt[pl.ds(b * LANES, LANES)] = zv
            return carry

        lax.fori_loop(0, OUT_FEAT * NREG, _zero_yt, 0)

        for q in range(2):
            pltpu.sync_copy(zeros,
                            shared2.at[q].at[pl.ds(nbase, NODES_PER_TILE)])
        plsc.subcore_barrier()

        def chain_body(r, carry):
            p = lax.rem(r, 2)
            buf = shared2.at[p]
            other = shared2.at[1 - p]

            def _chunk(c, carry2):
                for j in range(CHUNK // LANES):
                    sl = pl.ds(j * LANES, LANES)
                    g = plsc.load_gather(cur, [src2[c, sl]])
                    prod2[c, sl] = g * val2[c, sl]
                pltpu.async_copy(prod2.at[c], buf.at[dst2.at[c]], sem,
                                 add=True)
                return carry2

            lax.fori_loop(0, CHUNKS_PER_TILE, _chunk, 0)

            def _drain(c, carry2):
                pltpu.make_async_copy(prod2.at[c], buf.at[dst2.at[c]],
                                      sem).wait()
                return carry2

            lax.fori_loop(0, CHUNKS_PER_TILE, _drain, 0)
            plsc.subcore_barrier()

            pltpu.sync_copy(buf, cur)

            o_lo = jnp.maximum(0, (r - 1) // 2)
            o_hi = jnp.minimum(OUT_FEAT - 1, r // 2)

            def _acc_tap(o_t):
                av = plsc.load_gather(
                    atab, [jnp.full((LANES,), r * OUT_FEAT + o_t, jnp.int32)])

                def _yacc(b, carry2):
                    sl = pl.ds(o_t * NODES_PER_TILE + b * LANES, LANES)
                    yt[sl] = yt[sl] + av * cur[pl.ds(nbase + b * LANES,
                                                     LANES)]
                    return carry2

                lax.fori_loop(0, NREG, _yacc, 0)

            _acc_tap(o_lo)

            @pl.when(o_hi != o_lo)
            def _():
                _acc_tap(o_hi)

            pltpu.sync_copy(zeros, other.at[pl.ds(nbase, NODES_PER_TILE)])
            plsc.subcore_barrier()
            return carry

        lax.fori_loop(0, K_CHAIN, chain_body, 0)

        iota = lax.iota(jnp.int32, LANES)
        for o in range(OUT_FEAT):
            def _tr(b, carry2, o=o):
                v = yt[pl.ds(o * NODES_PER_TILE + b * LANES, LANES)]
                plsc.store_scatter(
                    ytr, [iota + b * LANES, jnp.full((LANES,), o, jnp.int32)],
                    v)
                return carry2

            lax.fori_loop(0, NREG, _tr, 0)

        @pl.when(cid == 0)
        def _():
            @pl.when(sid < NSUB - 1)
            def _():
                pltpu.sync_copy(ytr,
                                y_hbm.at[pl.ds(nbase, NODES_PER_TILE), :])

            @pl.when(sid == NSUB - 1)
            def _():
                last = N_NODES - (NSUB - 1) * NODES_PER_TILE
                pltpu.sync_copy(
                    ytr.at[pl.ds(0, last), :],
                    y_hbm.at[pl.ds((NSUB - 1) * NODES_PER_TILE, last), :])

    return chain


_chain_kernel = _sc_chain()


def kernel(x, edge_src, edge_dst, edge_val, filterCoeff):
    x = x.astype(jnp.float32)
    src = edge_src.astype(jnp.int32)
    dst = edge_dst.astype(jnp.int32)
    val = edge_val.astype(jnp.float32)
    coeff = filterCoeff.astype(jnp.float32)

    s_pad = jnp.pad(_rowsum(x), (0, N_PAD - N_NODES))

    pad = E_PAD - N_EDGES
    src2 = jnp.pad(src, (0, pad)).reshape(NSUB * CHUNKS_PER_TILE, CHUNK)
    dst2 = jnp.pad(dst, (0, pad)).reshape(NSUB * CHUNKS_PER_TILE, CHUNK)
    val2 = jnp.pad(val, (0, pad)).reshape(NSUB * CHUNKS_PER_TILE, CHUNK)

    r = jnp.arange(K_CHAIN)[:, None]
    o = jnp.arange(OUT_FEAT)[None, :]
    i = r - 2 * o
    valid = (i >= 0) & (i < FILTER_ORDER)
    atab = jnp.where(valid, coeff.T[jnp.clip(i, 0, FILTER_ORDER - 1), o], 0.0)
    atab = jnp.pad(atab.reshape(-1), (0, A_PAD - K_CHAIN * OUT_FEAT))

    return _chain_kernel(s_pad, src2, dst2, val2, atab)
